# baseline (speedup 1.0000x reference)
import functools

import numpy as np
import jax
import jax.numpy as jnp
from jax import lax
from jax.experimental import pallas as pl
from jax.experimental.pallas import tpu as pltpu
from jax.experimental.pallas import tpu_sc as plsc

_VOCAB = 1000000
_DIM = 64
_DEGREE = 3
_NUM_CTRL = 64
_B = 16384

_NC = 2
_NS = 16
_NW = _NC * _NS
_BPW = _B // _NW
_NG = _BPW // 16


def _sc_emb_body(ids_hbm, table_hbm, emb_out, idx_v, row_v, rows_v, sem_e):
    wid = lax.axis_index("s") * _NC + lax.axis_index("c")
    base = wid * _BPW
    for i in range(_NG):
        pltpu.sync_copy(ids_hbm.at[pl.ds(base + i * 16, 16)], idx_v.at[i])
    for i in range(_NG):
        v = idx_v[i]
        row_v[i] = jnp.where(v < _SPLIT, v, v - _SPLIT)
    copies = []
    for i in range(_NG):
        copies.append(pltpu.async_copy(
            table_hbm.at[row_v.at[i]],
            rows_v.at[pl.ds(i * 16, 16)], sem_e))
    for c in copies:
        c.wait()
    pltpu.sync_copy(rows_v, emb_out.at[pl.ds(base, _BPW)])


@functools.cache
def _sc_emb():
    return pl.kernel(
        _sc_emb_body,
        out_type=jax.ShapeDtypeStruct((_B, 2 * _DIM), jnp.float32),
        mesh=plsc.VectorSubcoreMesh(core_axis_name="c", subcore_axis_name="s"),
        scratch_types=(
            pltpu.VMEM((_NG, 16), jnp.int32),
            pltpu.VMEM((_NG, 16), jnp.int32),
            pltpu.VMEM((_BPW, 2 * _DIM), jnp.float32),
            pltpu.SemaphoreType.DMA,
        ),
        compiler_params=pltpu.CompilerParams(use_tc_tiling_on_sc=True,
                                             needs_layout_passes=False),
    )


def _sc_lin_body(ids_hbm, lin_hbm, lin_out, idx_v, linblk_v, lin_v, sem_l):
    wid = lax.axis_index("s") * _NC + lax.axis_index("c")
    base = wid * _BPW
    for i in range(_NG):
        pltpu.sync_copy(ids_hbm.at[pl.ds(base + i * 16, 16)], idx_v.at[i])
    lane = lax.iota(jnp.int32, 16)

    def lin_group(it, carry):
        ids16 = idx_v[it]
        copies = []
        for k in range(16):
            colb = pl.multiple_of(
                lax.bitwise_and(ids16[k], jnp.int32(~15)), 16)
            copies.append(pltpu.async_copy(
                lin_hbm.at[pl.ds(colb, 16), :], linblk_v.at[k], sem_l))
        for c in copies:
            c.wait()
        col = lax.bitwise_and(ids16, jnp.full((16,), 15, jnp.int32))
        zero = jnp.full((16,), 0, jnp.int32)
        lin_v[pl.ds(it * 16, 16)] = plsc.load_gather(
            linblk_v, [lane, col, zero])
        return carry

    lax.fori_loop(0, _NG, lin_group, 0)
    pltpu.sync_copy(lin_v, lin_out.at[pl.ds(base, _BPW)])


@functools.cache
def _sc_lin():
    return pl.kernel(
        _sc_lin_body,
        out_type=jax.ShapeDtypeStruct((_B,), jnp.float32),
        mesh=plsc.VectorSubcoreMesh(core_axis_name="c", subcore_axis_name="s"),
        scratch_types=(
            pltpu.VMEM((_NG, 16), jnp.int32),
            pltpu.VMEM((16, 16, 1), jnp.float32),
            pltpu.VMEM((_BPW,), jnp.float32),
            pltpu.SemaphoreType.DMA,
        ),
        compiler_params=pltpu.CompilerParams(use_tc_tiling_on_sc=False,
                                             needs_layout_passes=False),
    )


_TCOLS = 2048
_SPLIT = 497664
_PROWS = 503808


def _tc_transpose_body(a_ref, b_ref, out_ref):
    out_ref[...] = jnp.concatenate([a_ref[...].T, b_ref[...].T], axis=1)


_tc_transpose = pl.pallas_call(
    _tc_transpose_body,
    grid=(_PROWS // _TCOLS,),
    in_specs=[
        pl.BlockSpec((_DIM, _TCOLS), lambda i: (0, i)),
        pl.BlockSpec((_DIM, _TCOLS), lambda i: (0, i + _SPLIT // _TCOLS)),
    ],
    out_specs=pl.BlockSpec((_TCOLS, 2 * _DIM), lambda i: (i, 0)),
    out_shape=jax.ShapeDtypeStruct((_PROWS, 2 * _DIM), jnp.float32),
)


_ROWS = 2048


def _knot(j):
    return jnp.clip((j - float(_DEGREE)) * (1.0 / (_NUM_CTRL - _DEGREE)),
                    0.0, 1.0)


def _tc_spline_body(scalar_ref, mask_ref, ids_ref, disc2_ref, disc_lin_ref,
                    ectrl_ref, lctrl_ref, emb_out_ref, lin_out_ref):
    x = scalar_ref[...]
    t = jnp.clip(jax.nn.sigmoid(x), 1e-6, 1.0 - 1e-6)
    j0 = lax.broadcasted_iota(
        jnp.int32, (1, _NUM_CTRL + _DEGREE), 1).astype(jnp.float32)
    basis = jnp.where((t >= _knot(j0)) & (t < _knot(j0 + 1.0)), 1.0, 0.0)
    for p in range(1, _DEGREE + 1):
        w = _NUM_CTRL + _DEGREE - p
        j = lax.broadcasted_iota(jnp.int32, (1, w), 1).astype(jnp.float32)
        k_i = _knot(j)
        k_ip = _knot(j + float(p))
        k_i1 = _knot(j + 1.0)
        k_ip1 = _knot(j + float(p + 1))
        d1 = k_ip - k_i
        d2 = k_ip1 - k_i1
        r1 = jnp.where(d1 > 0, 1.0 / jnp.where(d1 > 0, d1, 1.0), 0.0)
        r2 = jnp.where(d2 > 0, 1.0 / jnp.where(d2 > 0, d2, 1.0), 0.0)
        basis = ((t - k_i) * r1 * basis[:, :-1]
                 + (k_ip1 - t) * r2 * basis[:, 1:])
    cont_emb = jnp.dot(basis, ectrl_ref[...],
                       preferred_element_type=jnp.float32)
    cont_lin = jnp.dot(basis, lctrl_ref[...],
                       preferred_element_type=jnp.float32)
    par = ids_ref[...] >= _SPLIT
    disc_emb = jnp.where(par, disc2_ref[:, _DIM:], disc2_ref[:, :_DIM])
    m = mask_ref[...] != 0
    emb_out_ref[...] = jnp.where(m, cont_emb, disc_emb)
    lin_out_ref[...] = jnp.where(m, cont_lin, disc_lin_ref[...])


_tc_spline = pl.pallas_call(
    _tc_spline_body,
    grid=(_B // _ROWS,),
    in_specs=[
        pl.BlockSpec((_ROWS, 1), lambda i: (i, 0)),
        pl.BlockSpec((_ROWS, 1), lambda i: (i, 0)),
        pl.BlockSpec((_ROWS, 1), lambda i: (i, 0)),
        pl.BlockSpec((_ROWS, 2 * _DIM), lambda i: (i, 0)),
        pl.BlockSpec((_ROWS, 1), lambda i: (i, 0)),
        pl.BlockSpec((_NUM_CTRL, _DIM), lambda i: (0, 0)),
        pl.BlockSpec((_NUM_CTRL, 1), lambda i: (0, 0)),
    ],
    out_specs=[
        pl.BlockSpec((_ROWS, _DIM), lambda i: (i, 0)),
        pl.BlockSpec((_ROWS, 1), lambda i: (i, 0)),
    ],
    out_shape=[
        jax.ShapeDtypeStruct((_B, _DIM), jnp.float32),
        jax.ShapeDtypeStruct((_B, 1), jnp.float32),
    ],
)


def kernel(token_ids, positive_mask, scalar, emb_table, lin_table,
           emb_ctrl, lin_ctrl):
    ids = token_ids.astype(jnp.int32)
    tab_t = emb_table.T
    pair_table = _tc_transpose(tab_t, tab_t)
    disc2 = _sc_emb()(ids, pair_table)
    disc_lin = _sc_lin()(ids, lin_table)
    emb, lin = _tc_spline(scalar[:, None],
                          positive_mask.astype(jnp.int32)[:, None],
                          ids[:, None], disc2, disc_lin[:, None],
                          emb_ctrl, lin_ctrl)
    return emb, lin[:, 0]


# speedup vs baseline: 1.9089x; 1.9089x over previous
import functools

import numpy as np
import jax
import jax.numpy as jnp
from jax import lax
from jax.experimental import pallas as pl
from jax.experimental.pallas import tpu as pltpu
from jax.experimental.pallas import tpu_sc as plsc

_VOCAB = 1000000
_DIM = 64
_DEGREE = 3
_NUM_CTRL = 64
_B = 16384

_NC = 2
_NS = 16
_NW = _NC * _NS
_BPW = _B // _NW
_CHUNK = 128
_NCH = _BPW // _CHUNK


def _sc_gather_body(ids_hbm, table_hbm, lin16_hbm, emb_out, lin_out,
                    idx_v, idx16_v, rows_v, lin16_v, lin_v, sem_e, sem_l):
    wid = lax.axis_index("s") * _NC + lax.axis_index("c")
    base = wid * _BPW
    for j in range(_NCH):
        pltpu.sync_copy(ids_hbm.at[pl.ds(base + j * _CHUNK, _CHUNK)],
                        idx_v.at[j])
    for j in range(_NCH):
        for o in range(_CHUNK // 16):
            ids16 = idx_v[j, pl.ds(o * 16, 16)]
            idx16_v[j, pl.ds(o * 16, 16)] = lax.shift_right_logical(ids16, 4)
    copies = []
    for j in range(_NCH):
        copies.append(pltpu.async_copy(
            table_hbm.at[idx_v.at[j]],
            rows_v.at[pl.ds(j * _CHUNK, _CHUNK)], sem_e))
        copies.append(pltpu.async_copy(
            lin16_hbm.at[idx16_v.at[j]],
            lin16_v.at[pl.ds(j * _CHUNK, _CHUNK)], sem_l))
    for c in copies:
        c.wait()
    lane = lax.iota(jnp.int32, 16)
    for it in range(_BPW // 16):
        ids16 = idx_v[it // 8, pl.ds((it % 8) * 16, 16)]
        row = jnp.full((16,), it * 16, jnp.int32) + lane
        col = lax.bitwise_and(ids16, jnp.full((16,), 15, jnp.int32))
        lin_v[pl.ds(it * 16, 16)] = plsc.load_gather(lin16_v, [row, col])
    pltpu.sync_copy(rows_v, emb_out.at[pl.ds(base, _BPW)])
    pltpu.sync_copy(lin_v, lin_out.at[pl.ds(base, _BPW)])


@functools.cache
def _sc_gather():
    return pl.kernel(
        _sc_gather_body,
        out_type=(jax.ShapeDtypeStruct((_B, _DIM), jnp.float32),
                  jax.ShapeDtypeStruct((_B,), jnp.float32)),
        mesh=plsc.VectorSubcoreMesh(core_axis_name="c", subcore_axis_name="s"),
        scratch_types=(
            pltpu.VMEM((_NCH, _CHUNK), jnp.int32),
            pltpu.VMEM((_NCH, _CHUNK), jnp.int32),
            pltpu.VMEM((_BPW, _DIM), jnp.float32),
            pltpu.VMEM((_BPW, 16), jnp.float32),
            pltpu.VMEM((_BPW,), jnp.float32),
            pltpu.SemaphoreType.DMA,
            pltpu.SemaphoreType.DMA,
        ),
        compiler_params=pltpu.CompilerParams(use_tc_tiling_on_sc=False,
                                             needs_layout_passes=False),
    )


_ROWS = 2048


def _knot(j):
    return jnp.clip((j - float(_DEGREE)) * (1.0 / (_NUM_CTRL - _DEGREE)),
                    0.0, 1.0)


def _tc_spline_body(scalar_ref, mask_ref, disc_emb_ref, disc_lin_ref,
                    ectrl_ref, lctrl_ref, emb_out_ref, lin_out_ref):
    x = scalar_ref[...]
    t = jnp.clip(jax.nn.sigmoid(x), 1e-6, 1.0 - 1e-6)
    j0 = lax.broadcasted_iota(
        jnp.int32, (1, _NUM_CTRL + _DEGREE), 1).astype(jnp.float32)
    basis = jnp.where((t >= _knot(j0)) & (t < _knot(j0 + 1.0)), 1.0, 0.0)
    for p in range(1, _DEGREE + 1):
        w = _NUM_CTRL + _DEGREE - p
        j = lax.broadcasted_iota(jnp.int32, (1, w), 1).astype(jnp.float32)
        k_i = _knot(j)
        k_ip = _knot(j + float(p))
        k_i1 = _knot(j + 1.0)
        k_ip1 = _knot(j + float(p + 1))
        d1 = k_ip - k_i
        d2 = k_ip1 - k_i1
        r1 = jnp.where(d1 > 0, 1.0 / jnp.where(d1 > 0, d1, 1.0), 0.0)
        r2 = jnp.where(d2 > 0, 1.0 / jnp.where(d2 > 0, d2, 1.0), 0.0)
        basis = ((t - k_i) * r1 * basis[:, :-1]
                 + (k_ip1 - t) * r2 * basis[:, 1:])
    cont_emb = jnp.dot(basis, ectrl_ref[...],
                       preferred_element_type=jnp.float32)
    cont_lin = jnp.dot(basis, lctrl_ref[...],
                       preferred_element_type=jnp.float32)
    m = mask_ref[...] != 0
    emb_out_ref[...] = jnp.where(m, cont_emb, disc_emb_ref[...])
    lin_out_ref[...] = jnp.where(m, cont_lin, disc_lin_ref[...])


_tc_spline = pl.pallas_call(
    _tc_spline_body,
    grid=(_B // _ROWS,),
    in_specs=[
        pl.BlockSpec((_ROWS, 1), lambda i: (i, 0)),
        pl.BlockSpec((_ROWS, 1), lambda i: (i, 0)),
        pl.BlockSpec((_ROWS, _DIM), lambda i: (i, 0)),
        pl.BlockSpec((_ROWS, 1), lambda i: (i, 0)),
        pl.BlockSpec((_NUM_CTRL, _DIM), lambda i: (0, 0)),
        pl.BlockSpec((_NUM_CTRL, 1), lambda i: (0, 0)),
    ],
    out_specs=[
        pl.BlockSpec((_ROWS, _DIM), lambda i: (i, 0)),
        pl.BlockSpec((_ROWS, 1), lambda i: (i, 0)),
    ],
    out_shape=[
        jax.ShapeDtypeStruct((_B, _DIM), jnp.float32),
        jax.ShapeDtypeStruct((_B, 1), jnp.float32),
    ],
)


def kernel(token_ids, positive_mask, scalar, emb_table, lin_table,
           emb_ctrl, lin_ctrl):
    ids = token_ids.astype(jnp.int32)
    lin16 = lin_table.reshape(_VOCAB // 16, 16)
    disc_emb, disc_lin = _sc_gather()(ids, emb_table, lin16)
    emb, lin = _tc_spline(scalar[:, None],
                          positive_mask.astype(jnp.int32)[:, None],
                          disc_emb, disc_lin[:, None], emb_ctrl, lin_ctrl)
    return emb, lin[:, 0]
